# baseline (speedup 1.0000x reference)
import functools

import jax
import jax.numpy as jnp
from jax import lax
from jax.experimental import pallas as pl
from jax.experimental.pallas import tpu as pltpu
from jax.experimental.pallas import tpu_sc as plsc

_EPS = 0.5
_N = 4096
_NC = 2
_NS = 16
_NW = _NC * _NS
_TC_ROWS = 2560
_SC_ROWS = _N - _TC_ROWS
_ROWS_PER_W = _SC_ROWS // _NW
_CHUNK = 8
_DEPTH = 2
_NCHUNK = _ROWS_PER_W // _CHUNK
_LANES = 16

_mesh = plsc.VectorSubcoreMesh(core_axis_name="c", subcore_axis_name="s")

_scratch = (
    [pltpu.VMEM((_CHUNK, _N), jnp.float32) for _ in range(_DEPTH)]
    + [pltpu.SemaphoreType.DMA for _ in range(2 * _DEPTH)]
)


@functools.partial(
    pl.kernel,
    out_type=jax.ShapeDtypeStruct((_SC_ROWS, _N), jnp.float32),
    mesh=_mesh,
    scratch_types=_scratch,
)
def _sc_mask(adj_hbm, out_hbm, *bufs_and_sems):
    bufs = bufs_and_sems[:_DEPTH]
    isems = bufs_and_sems[_DEPTH : 2 * _DEPTH]
    osems = bufs_and_sems[2 * _DEPTH :]

    wid = lax.axis_index("s") * _NC + lax.axis_index("c")
    base = _TC_ROWS + wid * _ROWS_PER_W

    def start_in(k):
        b = k % _DEPTH
        return pltpu.async_copy(
            adj_hbm.at[pl.ds(base + k * _CHUNK, _CHUNK)], bufs[b], isems[b]
        )

    def compute(b):
        def body(j, carry):
            c0 = j * _LANES
            for r in range(_CHUNK):
                v = bufs[b][r, pl.ds(c0, _LANES)]
                bufs[b][r, pl.ds(c0, _LANES)] = jnp.where(v > _EPS, v, 0.0)
            return carry

        lax.fori_loop(0, _N // _LANES, body, 0)

    cp_in = [start_in(k) for k in range(min(_DEPTH, _NCHUNK))]
    pending_out = [None] * _DEPTH
    for k in range(_NCHUNK):
        b = k % _DEPTH
        cp_in[b].wait()
        compute(b)
        pending_out[b] = pltpu.async_copy(
            bufs[b],
            out_hbm.at[pl.ds(wid * _ROWS_PER_W + k * _CHUNK, _CHUNK)],
            osems[b],
        )
        if k + _DEPTH < _NCHUNK:
            pending_out[b].wait()
            pending_out[b] = None
            cp_in[b] = start_in(k + _DEPTH)
    for b in range(_DEPTH):
        if pending_out[b] is not None:
            pending_out[b].wait()


def _tc_body(x_ref, o_ref):
    x = x_ref[...]
    o_ref[...] = jnp.where(x > _EPS, x, 0.0)


def kernel(adj):
    tc_out = pl.pallas_call(
        _tc_body,
        out_shape=jax.ShapeDtypeStruct((_TC_ROWS, _N), jnp.float32),
        grid=(_TC_ROWS // 512,),
        in_specs=[pl.BlockSpec((512, _N), lambda i: (i, 0))],
        out_specs=pl.BlockSpec((512, _N), lambda i: (i, 0)),
    )(adj)
    sc_out = _sc_mask(adj)
    return jnp.concatenate([tc_out, sc_out], axis=0)
